# Initial kernel scaffold; baseline (speedup 1.0000x reference)
import jax
import jax.numpy as jnp
from jax.experimental import pallas as pl


def kernel(branch_points, mid_points, branch_labels, mid_labels, pe_gauss, branch_table, mid_table):
    raise NotImplementedError("write your pallas kernel here")


# speedup vs baseline: 2.5825x; 2.5825x over previous
import functools

import jax
import jax.numpy as jnp
import numpy as np
from jax.experimental import pallas as pl
from jax.experimental.pallas import tpu as pltpu

IMG_SIZE = 1024.0
NPTS = 250
PCH = 256
PHALF = 128


def _body(x_ref, y_ref, lab_ref, g_ref, tab_ref, out_ref):
    bb = x_ref.shape[0]
    cx = x_ref[...] * (2.0 / IMG_SIZE) - 1.0
    cy = y_ref[...] * (2.0 / IMG_SIZE) - 1.0
    g0 = g_ref[0:1, :]
    g1 = g_ref[1:2, :]
    two_pi = np.float32(2.0 * np.pi)
    phase = two_pi * (cx[..., None] * g0[None] + cy[..., None] * g1[None])
    pe = jnp.concatenate([jnp.sin(phase), jnp.cos(phase)], axis=-1)

    lab = lab_ref[...]
    onehot = (lab[..., None] ==
              jax.lax.broadcasted_iota(jnp.int32, (bb, NPTS, 32), 2)
              ).astype(jnp.float32)
    emb = jax.lax.dot_general(
        onehot.reshape(bb * NPTS, 32), tab_ref[...],
        dimension_numbers=(((1,), (0,)), ((), ())),
        preferred_element_type=jnp.float32,
    ).reshape(bb, NPTS, PCH)
    out_ref[...] = pe + emb


@functools.partial(jax.jit, static_argnames=("block_b",))
def _run(x, y, labels, pe_gauss, table, block_b=8):
    B = x.shape[0]
    grid = (B // block_b,)
    return pl.pallas_call(
        _body,
        grid=grid,
        in_specs=[
            pl.BlockSpec((block_b, NPTS), lambda i: (i, 0)),
            pl.BlockSpec((block_b, NPTS), lambda i: (i, 0)),
            pl.BlockSpec((block_b, NPTS), lambda i: (i, 0)),
            pl.BlockSpec((2, PHALF), lambda i: (0, 0)),
            pl.BlockSpec((32, PCH), lambda i: (0, 0)),
        ],
        out_specs=pl.BlockSpec((block_b, NPTS, PCH), lambda i: (i, 0, 0)),
        out_shape=jax.ShapeDtypeStruct((B, NPTS, PCH), jnp.float32),
    )(x, y, labels, pe_gauss, table)


def kernel(branch_points, mid_points, branch_labels, mid_labels, pe_gauss,
           branch_table, mid_table):
    pts = jnp.concatenate([branch_points, mid_points], axis=1)
    x = pts[..., 0]
    y = pts[..., 1]
    labels = jnp.concatenate(
        [branch_labels, mid_labels + 16], axis=1).astype(jnp.int32)
    table = jnp.concatenate([branch_table, mid_table], axis=0)
    return _run(x, y, labels, pe_gauss, table)
